# baseline (speedup 1.0000x reference)
import functools

import jax
import jax.numpy as jnp
from jax import lax
from jax.experimental import pallas as pl
from jax.experimental.pallas import tpu as pltpu
from jax.experimental.pallas import tpu_sc as plsc

T = 4
N = 10000
E = 320000
H = 128
ED = 64
G = 64

NC = 2
NS = 16
NW = NC * NS

C = 64
NB = 5
EPW = 10240
EPAD = NW * EPW
NCHUNK = EPW // C
NP = 10240
ZR = 64
RPT_Z = NP // NS // ZR
RPT_OUT = NP // NS

def _edge_agg_body(idx_hbm, h_hbm, out_hbm, idx_bufs, rows_bufs, agg_sh,
                   isems, gsems, ssems, zsem):
    cid = lax.axis_index("c")
    sid = lax.axis_index("s")
    wid = cid * NS + sid

    rows_a = rows_bufs[0]

    def _zrow(i, carry):
        for j in range(H // 16):
            rows_a[i, pl.ds(j * 16, 16)] = jnp.zeros((16,), jnp.float32)
        return carry

    lax.fori_loop(0, C, _zrow, 0)

    zbase = sid * RPT_OUT
    for i in range(RPT_OUT // C):
        pltpu.async_copy(rows_a, agg_sh.at[pl.ds(zbase + i * C, C)], zsem)

    for b in range(NB - 1):
        pltpu.async_copy(idx_hbm.at[wid, b], idx_bufs[b], isems[b])

    for i in range(RPT_OUT // C):
        pltpu.make_async_copy(rows_a, agg_sh.at[pl.ds(zbase + i * C, C)],
                              zsem).wait()
    plsc.subcore_barrier()

    for b in range(NB - 2):
        pltpu.make_async_copy(idx_hbm.at[wid, b], idx_bufs[b],
                              isems[b]).wait()
        pltpu.async_copy(h_hbm.at[idx_bufs[b].at[0]], rows_bufs[b], gsems[b])

    def _group(i, carry):
        for b in range(NB):
            c = i * NB + b
            pb = (b + NB - 1) % NB
            gb = (b + NB - 2) % NB

            @pl.when(c >= 1)
            def _():
                pltpu.make_async_copy(rows_bufs[pb],
                                      agg_sh.at[idx_bufs[pb].at[1]],
                                      ssems[pb]).wait()

            @pl.when(c + NB - 1 < NCHUNK)
            def _():
                pltpu.async_copy(idx_hbm.at[wid, c + NB - 1], idx_bufs[pb],
                                 isems[pb])

            @pl.when(c + NB - 2 < NCHUNK)
            def _():
                pltpu.make_async_copy(idx_hbm.at[wid, 0], idx_bufs[gb],
                                      isems[gb]).wait()
                pltpu.async_copy(h_hbm.at[idx_bufs[gb].at[0]], rows_bufs[gb],
                                 gsems[gb])

            pltpu.make_async_copy(h_hbm.at[idx_bufs[b].at[0]], rows_bufs[b],
                                  gsems[b]).wait()
            pltpu.async_copy(rows_bufs[b], agg_sh.at[idx_bufs[b].at[1]],
                             ssems[b], add=True)

        return carry

    lax.fori_loop(0, NCHUNK // NB, _group, 0)
    lb = (NCHUNK - 1) % NB
    pltpu.make_async_copy(rows_bufs[lb], agg_sh.at[idx_bufs[lb].at[1]],
                          ssems[lb]).wait()
    plsc.subcore_barrier()

    obase = sid * RPT_OUT
    pltpu.sync_copy(agg_sh.at[pl.ds(obase, RPT_OUT)],
                    out_hbm.at[cid, pl.ds(obase, RPT_OUT)])


@functools.cache
def _build_edge_agg():
    mesh = plsc.VectorSubcoreMesh(core_axis_name="c", subcore_axis_name="s")
    return pl.kernel(
        _edge_agg_body,
        out_type=jax.ShapeDtypeStruct((NC, NP, H), jnp.float32),
        mesh=mesh,
        scratch_types=[
            [pltpu.VMEM((2, C), jnp.int32) for _ in range(NB)],
            [pltpu.VMEM((C, H), jnp.float32) for _ in range(NB)],
            pltpu.VMEM_SHARED((NP, H), jnp.float32),
            [pltpu.SemaphoreType.DMA for _ in range(NB)],
            [pltpu.SemaphoreType.DMA for _ in range(NB)],
            [pltpu.SemaphoreType.DMA for _ in range(NB)],
            pltpu.SemaphoreType.DMA,
        ],
    )


def _edge_agg(idx, h):
    return _build_edge_agg()(idx, h)


R = 1000
GRID = N // R


def _gru_math(agg_ref, h_ref, A_ref, Wzrh_ref, Uzr_ref, Uh_ref, bzrh_ref):
    dot = functools.partial(jnp.dot, preferred_element_type=jnp.float32)
    agg = agg_ref[0] + agg_ref[1]
    hh = h_ref[...]
    m = dot(agg, A_ref[...])
    mW = dot(m, Wzrh_ref[...]) + bzrh_ref[...]
    hU = dot(hh, Uzr_ref[...])
    z = jax.nn.sigmoid(mW[:, 0:H] + hU[:, 0:H])
    r = jax.nn.sigmoid(mW[:, H:2 * H] + hU[:, H:2 * H])
    ht = jnp.tanh(mW[:, 2 * H:3 * H] + dot(r * hh, Uh_ref[...]))
    return (1.0 - z) * hh + z * ht


def _gru_body(agg_ref, h_ref, A_ref, Wzrh_ref, Uzr_ref, Uh_ref, bzrh_ref,
              out_ref):
    out_ref[...] = _gru_math(agg_ref, h_ref, A_ref, Wzrh_ref, Uzr_ref,
                             Uh_ref, bzrh_ref)


_GRU_SPECS = [
    pl.BlockSpec((NC, R, H), lambda i: (0, i, 0)),
    pl.BlockSpec((R, H), lambda i: (i, 0)),
    pl.BlockSpec((H, H), lambda i: (0, 0)),
    pl.BlockSpec((H, 3 * H), lambda i: (0, 0)),
    pl.BlockSpec((H, 2 * H), lambda i: (0, 0)),
    pl.BlockSpec((H, H), lambda i: (0, 0)),
    pl.BlockSpec((1, 3 * H), lambda i: (0, 0)),
]


def _gru_round(agg, h, A, Wzrh, Uzr, Uh, bzrh):
    return pl.pallas_call(
        _gru_body,
        grid=(GRID,),
        in_specs=_GRU_SPECS,
        out_specs=pl.BlockSpec((R, H), lambda i: (i, 0)),
        out_shape=jax.ShapeDtypeStruct((N, H), jnp.float32),
    )(agg, h, A, Wzrh, Uzr, Uh, bzrh)


def _gru_readout_body(agg_ref, h_ref, A_ref, Wzrh_ref, Uzr_ref, Uh_ref,
                      bzrh_ref, ids_ref, Wup_ref, bup_ref, Wgate_ref,
                      bgate_ref, out_ref):
    dot = functools.partial(jnp.dot, preferred_element_type=jnp.float32)
    hn = _gru_math(agg_ref, h_ref, A_ref, Wzrh_ref, Uzr_ref, Uh_ref,
                   bzrh_ref)
    proj = dot(hn, Wup_ref[...]) + bup_ref[...]
    gate = jax.nn.sigmoid(dot(hn, Wgate_ref[...]) + bgate_ref[...])
    gated = gate * proj
    ids = ids_ref[0, 0, :].reshape(1, R)
    ohT = (lax.broadcasted_iota(jnp.int32, (G, R), 0) == ids).astype(
        jnp.float32)
    contrib = dot(ohT, gated)

    @pl.when(pl.program_id(0) == 0)
    def _():
        out_ref[...] = jnp.zeros_like(out_ref)

    out_ref[...] += contrib


def _gru_readout(agg, h, A, Wzrh, Uzr, Uh, bzrh, ids3d, Wup, bup, Wgate,
                 bgate):
    return pl.pallas_call(
        _gru_readout_body,
        grid=(GRID,),
        in_specs=_GRU_SPECS + [
            pl.BlockSpec((1, 1, R), lambda i: (i, 0, 0)),
            pl.BlockSpec((H, ED), lambda i: (0, 0)),
            pl.BlockSpec((1, ED), lambda i: (0, 0)),
            pl.BlockSpec((H, ED), lambda i: (0, 0)),
            pl.BlockSpec((1, ED), lambda i: (0, 0)),
        ],
        out_specs=pl.BlockSpec((G, ED), lambda i: (0, 0)),
        out_shape=jax.ShapeDtypeStruct((G, ED), jnp.float32),
    )(agg, h, A, Wzrh, Uzr, Uh, bzrh, ids3d, Wup, bup, Wgate, bgate)


def kernel(node_features, edge_index, node_to_graph_id, A, Wz, Uz, bz, Wr,
           Ur, br, Wh, Uh, bh, Wup, bup, Wgate, bgate):
    src = edge_index[0].astype(jnp.int32)
    dst = edge_index[1].astype(jnp.int32)
    npad = EPAD - E
    pad_i = jnp.arange(npad, dtype=jnp.int32)
    src = jnp.concatenate([src, pad_i % N])
    dst = jnp.concatenate([dst, N + pad_i % (NP - N)])
    idx = jnp.stack([src.reshape(NW, NCHUNK, C),
                     dst.reshape(NW, NCHUNK, C)], axis=2)
    ids3d = node_to_graph_id.astype(jnp.int32).reshape(GRID, 1, R)
    Wzrh = jnp.concatenate([Wz, Wr, Wh], axis=1)
    Uzr = jnp.concatenate([Uz, Ur], axis=1)
    bzrh = jnp.concatenate([bz, br, bh]).reshape(1, 3 * H)
    bup2, bgate2 = bup.reshape(1, ED), bgate.reshape(1, ED)

    h = node_features
    for _ in range(T - 1):
        agg = _edge_agg(idx, h)
        h = _gru_round(agg, h, A, Wzrh, Uzr, Uh, bzrh)
    agg = _edge_agg(idx, h)
    return _gru_readout(agg, h, A, Wzrh, Uzr, Uh, bzrh, ids3d, Wup, bup2,
                        Wgate, bgate2)


# speedup vs baseline: 1.0207x; 1.0207x over previous
import functools

import jax
import jax.numpy as jnp
from jax import lax
from jax.experimental import pallas as pl
from jax.experimental.pallas import tpu as pltpu
from jax.experimental.pallas import tpu_sc as plsc

T = 4
N = 10000
E = 320000
H = 128
ED = 64
G = 64

NC = 2
NS = 16
NW = NC * NS

C = 80
NB = 4
EPW = 10240
EPAD = NW * EPW
NCHUNK = EPW // C
NP = 10240
ZR = 64
RPT_Z = NP // NS // ZR
RPT_OUT = NP // NS

def _edge_agg_body(idx_hbm, h_hbm, out_hbm, idx_bufs, rows_bufs, agg_sh,
                   isems, gsems, ssems, zsem):
    cid = lax.axis_index("c")
    sid = lax.axis_index("s")
    wid = cid * NS + sid

    rows_a = rows_bufs[0]

    def _zrow(i, carry):
        for j in range(H // 16):
            rows_a[i, pl.ds(j * 16, 16)] = jnp.zeros((16,), jnp.float32)
        return carry

    lax.fori_loop(0, C, _zrow, 0)

    zbase = sid * RPT_OUT
    for i in range(RPT_OUT // C):
        pltpu.async_copy(rows_a, agg_sh.at[pl.ds(zbase + i * C, C)], zsem)

    for b in range(NB - 1):
        pltpu.async_copy(idx_hbm.at[wid, b], idx_bufs[b], isems[b])

    for i in range(RPT_OUT // C):
        pltpu.make_async_copy(rows_a, agg_sh.at[pl.ds(zbase + i * C, C)],
                              zsem).wait()
    plsc.subcore_barrier()

    for b in range(NB - 2):
        pltpu.make_async_copy(idx_hbm.at[wid, b], idx_bufs[b],
                              isems[b]).wait()
        pltpu.async_copy(h_hbm.at[idx_bufs[b].at[0]], rows_bufs[b], gsems[b])

    def _group(i, carry):
        for b in range(NB):
            c = i * NB + b
            pb = (b + NB - 1) % NB
            gb = (b + NB - 2) % NB

            @pl.when(c >= 1)
            def _():
                pltpu.make_async_copy(rows_bufs[pb],
                                      agg_sh.at[idx_bufs[pb].at[1]],
                                      ssems[pb]).wait()

            @pl.when(c + NB - 1 < NCHUNK)
            def _():
                pltpu.async_copy(idx_hbm.at[wid, c + NB - 1], idx_bufs[pb],
                                 isems[pb])

            @pl.when(c + NB - 2 < NCHUNK)
            def _():
                pltpu.make_async_copy(idx_hbm.at[wid, 0], idx_bufs[gb],
                                      isems[gb]).wait()
                pltpu.async_copy(h_hbm.at[idx_bufs[gb].at[0]], rows_bufs[gb],
                                 gsems[gb])

            pltpu.make_async_copy(h_hbm.at[idx_bufs[b].at[0]], rows_bufs[b],
                                  gsems[b]).wait()
            pltpu.async_copy(rows_bufs[b], agg_sh.at[idx_bufs[b].at[1]],
                             ssems[b], add=True)

        return carry

    lax.fori_loop(0, NCHUNK // NB, _group, 0)
    lb = (NCHUNK - 1) % NB
    pltpu.make_async_copy(rows_bufs[lb], agg_sh.at[idx_bufs[lb].at[1]],
                          ssems[lb]).wait()
    plsc.subcore_barrier()

    obase = sid * RPT_OUT
    pltpu.sync_copy(agg_sh.at[pl.ds(obase, RPT_OUT)],
                    out_hbm.at[cid, pl.ds(obase, RPT_OUT)])


@functools.cache
def _build_edge_agg():
    mesh = plsc.VectorSubcoreMesh(core_axis_name="c", subcore_axis_name="s")
    return pl.kernel(
        _edge_agg_body,
        out_type=jax.ShapeDtypeStruct((NC, NP, H), jnp.float32),
        mesh=mesh,
        scratch_types=[
            [pltpu.VMEM((2, C), jnp.int32) for _ in range(NB)],
            [pltpu.VMEM((C, H), jnp.float32) for _ in range(NB)],
            pltpu.VMEM_SHARED((NP, H), jnp.float32),
            [pltpu.SemaphoreType.DMA for _ in range(NB)],
            [pltpu.SemaphoreType.DMA for _ in range(NB)],
            [pltpu.SemaphoreType.DMA for _ in range(NB)],
            pltpu.SemaphoreType.DMA,
        ],
    )


def _edge_agg(idx, h):
    return _build_edge_agg()(idx, h)


R = 1000
GRID = N // R


def _gru_math(agg_ref, h_ref, A_ref, Wzrh_ref, Uzr_ref, Uh_ref, bzrh_ref):
    dot = functools.partial(jnp.dot, preferred_element_type=jnp.float32)
    agg = agg_ref[0] + agg_ref[1]
    hh = h_ref[...]
    m = dot(agg, A_ref[...])
    mW = dot(m, Wzrh_ref[...]) + bzrh_ref[...]
    hU = dot(hh, Uzr_ref[...])
    z = jax.nn.sigmoid(mW[:, 0:H] + hU[:, 0:H])
    r = jax.nn.sigmoid(mW[:, H:2 * H] + hU[:, H:2 * H])
    ht = jnp.tanh(mW[:, 2 * H:3 * H] + dot(r * hh, Uh_ref[...]))
    return (1.0 - z) * hh + z * ht


def _gru_body(agg_ref, h_ref, A_ref, Wzrh_ref, Uzr_ref, Uh_ref, bzrh_ref,
              out_ref):
    out_ref[...] = _gru_math(agg_ref, h_ref, A_ref, Wzrh_ref, Uzr_ref,
                             Uh_ref, bzrh_ref)


_GRU_SPECS = [
    pl.BlockSpec((NC, R, H), lambda i: (0, i, 0)),
    pl.BlockSpec((R, H), lambda i: (i, 0)),
    pl.BlockSpec((H, H), lambda i: (0, 0)),
    pl.BlockSpec((H, 3 * H), lambda i: (0, 0)),
    pl.BlockSpec((H, 2 * H), lambda i: (0, 0)),
    pl.BlockSpec((H, H), lambda i: (0, 0)),
    pl.BlockSpec((1, 3 * H), lambda i: (0, 0)),
]


def _gru_round(agg, h, A, Wzrh, Uzr, Uh, bzrh):
    return pl.pallas_call(
        _gru_body,
        grid=(GRID,),
        in_specs=_GRU_SPECS,
        out_specs=pl.BlockSpec((R, H), lambda i: (i, 0)),
        out_shape=jax.ShapeDtypeStruct((N, H), jnp.float32),
    )(agg, h, A, Wzrh, Uzr, Uh, bzrh)


def _gru_readout_body(agg_ref, h_ref, A_ref, Wzrh_ref, Uzr_ref, Uh_ref,
                      bzrh_ref, ids_ref, Wup_ref, bup_ref, Wgate_ref,
                      bgate_ref, out_ref):
    dot = functools.partial(jnp.dot, preferred_element_type=jnp.float32)
    hn = _gru_math(agg_ref, h_ref, A_ref, Wzrh_ref, Uzr_ref, Uh_ref,
                   bzrh_ref)
    proj = dot(hn, Wup_ref[...]) + bup_ref[...]
    gate = jax.nn.sigmoid(dot(hn, Wgate_ref[...]) + bgate_ref[...])
    gated = gate * proj
    ids = ids_ref[0, 0, :].reshape(1, R)
    ohT = (lax.broadcasted_iota(jnp.int32, (G, R), 0) == ids).astype(
        jnp.float32)
    contrib = dot(ohT, gated)

    @pl.when(pl.program_id(0) == 0)
    def _():
        out_ref[...] = jnp.zeros_like(out_ref)

    out_ref[...] += contrib


def _gru_readout(agg, h, A, Wzrh, Uzr, Uh, bzrh, ids3d, Wup, bup, Wgate,
                 bgate):
    return pl.pallas_call(
        _gru_readout_body,
        grid=(GRID,),
        in_specs=_GRU_SPECS + [
            pl.BlockSpec((1, 1, R), lambda i: (i, 0, 0)),
            pl.BlockSpec((H, ED), lambda i: (0, 0)),
            pl.BlockSpec((1, ED), lambda i: (0, 0)),
            pl.BlockSpec((H, ED), lambda i: (0, 0)),
            pl.BlockSpec((1, ED), lambda i: (0, 0)),
        ],
        out_specs=pl.BlockSpec((G, ED), lambda i: (0, 0)),
        out_shape=jax.ShapeDtypeStruct((G, ED), jnp.float32),
    )(agg, h, A, Wzrh, Uzr, Uh, bzrh, ids3d, Wup, bup, Wgate, bgate)


def kernel(node_features, edge_index, node_to_graph_id, A, Wz, Uz, bz, Wr,
           Ur, br, Wh, Uh, bh, Wup, bup, Wgate, bgate):
    src = edge_index[0].astype(jnp.int32)
    dst = edge_index[1].astype(jnp.int32)
    npad = EPAD - E
    pad_i = jnp.arange(npad, dtype=jnp.int32)
    src = jnp.concatenate([src, pad_i % N])
    dst = jnp.concatenate([dst, N + pad_i % (NP - N)])
    idx = jnp.stack([src.reshape(NW, NCHUNK, C),
                     dst.reshape(NW, NCHUNK, C)], axis=2)
    ids3d = node_to_graph_id.astype(jnp.int32).reshape(GRID, 1, R)
    Wzrh = jnp.concatenate([Wz, Wr, Wh], axis=1)
    Uzr = jnp.concatenate([Uz, Ur], axis=1)
    bzrh = jnp.concatenate([bz, br, bh]).reshape(1, 3 * H)
    bup2, bgate2 = bup.reshape(1, ED), bgate.reshape(1, ED)

    h = node_features
    for _ in range(T - 1):
        agg = _edge_agg(idx, h)
        h = _gru_round(agg, h, A, Wzrh, Uzr, Uh, bzrh)
    agg = _edge_agg(idx, h)
    return _gru_readout(agg, h, A, Wzrh, Uzr, Uh, bzrh, ids3d, Wup, bup2,
                        Wgate, bgate2)
